# baseline (speedup 1.0000x reference)
import functools

import jax
import jax.numpy as jnp
from jax import lax
from jax.experimental import pallas as pl
from jax.experimental.pallas import tpu as pltpu
from jax.experimental.pallas import tpu_sc as plsc

_EMB = 32
_B = 16384
_NROWS = 1000000
_PACK = 4
_PROWS = _NROWS // _PACK
_NUMF = 16
_HID = 128
_NCLS = 2
_NC, _NS = 2, 16
_NW = _NC * _NS
_BPW = _B // _NW
_CH = 128
_NCH = _BPW // _CH
_LANES = 16


def _gather_body(sidx_hbm, tidx_hbm, src4, tgt4, souT, touT,
                 sidx_v, tidx_v, srow, trow, sph, tph,
                 sbufs, tbufs, soutT, toutT, sem):
    wid = lax.axis_index("s") * _NC + lax.axis_index("c")
    base = pl.multiple_of(wid * _BPW, 8)
    pltpu.sync_copy(sidx_hbm.at[pl.ds(base, _BPW)], sidx_v)
    pltpu.sync_copy(tidx_hbm.at[pl.ds(base, _BPW)], tidx_v)

    for idx_v, row_v, ph_v in ((sidx_v, srow, sph), (tidx_v, trow, tph)):
        for k in range(_BPW // _LANES):
            sl = pl.ds(k * _LANES, _LANES)
            i = idx_v[sl]
            row_v[sl] = i >> 2
            ph_v[sl] = i & 3

    def fire(j):
        isl = pl.ds(j * _CH, _CH)
        return (pltpu.async_copy(src4.at[srow.at[isl]], sbufs.at[j % 2], sem),
                pltpu.async_copy(tgt4.at[trow.at[isl]], tbufs.at[j % 2], sem))

    inflight = {0: fire(0), 1: fire(1)}
    for j in range(_NCH):
        cs, ct = inflight.pop(j)
        cs.wait()
        ct.wait()
        for buf_pair, ph_v, outT in ((sbufs, sph, soutT),
                                     (tbufs, tph, toutT)):
            rows_v = buf_pair.at[j % 2]

            def extract(k, carry, rows_v=rows_v, ph_v=ph_v, outT=outT, j=j):
                rid = jax.lax.broadcasted_iota(jnp.int32, (_LANES,), 0) \
                    + k * _LANES
                ph = plsc.load_gather(ph_v, [j * _CH + rid])
                lane0 = ph * _EMB
                for d in range(_EMB):
                    vals = plsc.load_gather(rows_v, [rid, lane0 + d])
                    outT[d, pl.ds(j * _CH + k * _LANES, _LANES)] = vals
                return carry

            lax.fori_loop(0, _CH // _LANES, extract, 0)
        if j + 2 < _NCH:
            inflight[j + 2] = fire(j + 2)

    pltpu.sync_copy(soutT, souT.at[:, pl.ds(base, _BPW)])
    pltpu.sync_copy(toutT, touT.at[:, pl.ds(base, _BPW)])


_gather_cache = []


def _gather(*args):
    if not _gather_cache:
        mesh = plsc.VectorSubcoreMesh(
            core_axis_name="c", subcore_axis_name="s",
            num_cores=_NC, num_subcores=_NS,
        )
        _gather_cache.append(pl.kernel(
            _gather_body,
            out_type=(
                jax.ShapeDtypeStruct((_EMB, _B), jnp.float32),
                jax.ShapeDtypeStruct((_EMB, _B), jnp.float32),
            ),
            mesh=mesh,
            scratch_types=[
                pltpu.VMEM((_BPW,), jnp.int32),
                pltpu.VMEM((_BPW,), jnp.int32),
                pltpu.VMEM((_BPW,), jnp.int32),
                pltpu.VMEM((_BPW,), jnp.int32),
                pltpu.VMEM((_BPW,), jnp.int32),
                pltpu.VMEM((_BPW,), jnp.int32),
                pltpu.VMEM((2, _CH, 128), jnp.float32),
                pltpu.VMEM((2, _CH, 128), jnp.float32),
                pltpu.VMEM((_EMB, _BPW), jnp.float32),
                pltpu.VMEM((_EMB, _BPW), jnp.float32),
                pltpu.SemaphoreType.DMA,
            ],
            compiler_params=pltpu.CompilerParams(needs_layout_passes=False),
        ))
    return _gather_cache[0](*args)


def _mlp_body(sT, tT, n, w1s, w1t, w1n, b1, w2, b2, o):
    cdim = (((0,), (0,)), ((), ()))
    h = (lax.dot_general(sT[...], w1s[...], cdim,
                         preferred_element_type=jnp.float32)
         + lax.dot_general(tT[...], w1t[...], cdim,
                           preferred_element_type=jnp.float32)
         + jnp.dot(n[...], w1n[...], preferred_element_type=jnp.float32)
         + b1[...])
    h = jnp.maximum(h, 0.0)
    o[...] = jnp.dot(h, w2[...], preferred_element_type=jnp.float32) + b2[...]


_BLK = 2048


def _mlp(sT, tT, n, w1s, w1t, w1n, b1, w2, b2):
    grid = (_B // _BLK,)
    full = lambda i: (0, 0)
    return pl.pallas_call(
        _mlp_body,
        grid=grid,
        in_specs=[
            pl.BlockSpec((_EMB, _BLK), lambda i: (0, i)),
            pl.BlockSpec((_EMB, _BLK), lambda i: (0, i)),
            pl.BlockSpec((_BLK, _NUMF), lambda i: (i, 0)),
            pl.BlockSpec((_EMB, _HID), full),
            pl.BlockSpec((_EMB, _HID), full),
            pl.BlockSpec((_NUMF, _HID), full),
            pl.BlockSpec((1, _HID), full),
            pl.BlockSpec((_HID, _NCLS), full),
            pl.BlockSpec((1, _NCLS), full),
        ],
        out_specs=pl.BlockSpec((_BLK, _NCLS), lambda i: (i, 0)),
        out_shape=jax.ShapeDtypeStruct((_B, _NCLS), jnp.float32),
    )(sT, tT, n, w1s, w1t, w1n, b1, w2, b2)


def kernel(cat_feats, num_feats, src_emb, tgt_emb, W1, b1, W2, b2):
    src_id = cat_feats[:, 0]
    tgt_id = cat_feats[:, 1]
    sT, tT = _gather(src_id, tgt_id,
                     src_emb.reshape(_PROWS, 128),
                     tgt_emb.reshape(_PROWS, 128))
    w1s = W1[:, :_EMB].T
    w1t = W1[:, _EMB:2 * _EMB].T
    w1n = W1[:, 2 * _EMB:].T
    return _mlp(sT, tT, num_feats, w1s, w1t, w1n,
                b1.reshape(1, _HID), W2.T, b2.reshape(1, _NCLS))


# speedup vs baseline: 1.0143x; 1.0143x over previous
import functools

import jax
import jax.numpy as jnp
from jax import lax
from jax.experimental import pallas as pl
from jax.experimental.pallas import tpu as pltpu
from jax.experimental.pallas import tpu_sc as plsc

_EMB = 32
_B = 16384
_NUMF = 16
_HID = 128
_NCLS = 2
_NC, _NS = 2, 16
_NW = _NC * _NS
_BPW = _B // _NW
_CH = 128
_NCH = _BPW // _CH

def _gather_body(src_id, tgt_id, src_emb, tgt_emb, src_out, tgt_out,
                 sidx, tidx, srows, trows, sem):
    wid = lax.axis_index("s") * _NC + lax.axis_index("c")
    base = pl.multiple_of(wid * _BPW, 8)
    pltpu.sync_copy(src_id.at[pl.ds(base, _BPW)], sidx)
    pltpu.sync_copy(tgt_id.at[pl.ds(base, _BPW)], tidx)
    copies = []
    for j in range(_NCH):
        isl = pl.ds(j * _CH, _CH)
        copies.append(pltpu.async_copy(src_emb.at[sidx.at[isl]], srows.at[j], sem))
        copies.append(pltpu.async_copy(tgt_emb.at[tidx.at[isl]], trows.at[j], sem))
    for c in copies:
        c.wait()
    pltpu.sync_copy(srows, src_out.at[wid])
    pltpu.sync_copy(trows, tgt_out.at[wid])


_gather_cache = []


def _gather(*args):
    if not _gather_cache:
        mesh = plsc.VectorSubcoreMesh(
            core_axis_name="c", subcore_axis_name="s",
            num_cores=_NC, num_subcores=_NS,
        )
        _gather_cache.append(pl.kernel(
            _gather_body,
            out_type=(
                jax.ShapeDtypeStruct((_NW, _NCH, _CH, _EMB), jnp.float32),
                jax.ShapeDtypeStruct((_NW, _NCH, _CH, _EMB), jnp.float32),
            ),
            mesh=mesh,
            scratch_types=[
                pltpu.VMEM((_BPW,), jnp.int32),
                pltpu.VMEM((_BPW,), jnp.int32),
                pltpu.VMEM((_NCH, _CH, _EMB), jnp.float32),
                pltpu.VMEM((_NCH, _CH, _EMB), jnp.float32),
                pltpu.SemaphoreType.DMA,
            ],
            compiler_params=pltpu.CompilerParams(use_tc_tiling_on_sc=False),
        ))
    return _gather_cache[0](*args)


def _mlp_body(s, t, n, w1s, w1t, w1n, b1, w2, b2, o):
    h = (jnp.dot(s[...], w1s[...], preferred_element_type=jnp.float32)
         + jnp.dot(t[...], w1t[...], preferred_element_type=jnp.float32)
         + jnp.dot(n[...], w1n[...], preferred_element_type=jnp.float32)
         + b1[...])
    h = jnp.maximum(h, 0.0)
    o[...] = jnp.dot(h, w2[...], preferred_element_type=jnp.float32) + b2[...]


_BLK = 2048


def _mlp(s, t, n, w1s, w1t, w1n, b1, w2, b2):
    grid = (_B // _BLK,)
    full = lambda i: (0, 0)
    return pl.pallas_call(
        _mlp_body,
        grid=grid,
        in_specs=[
            pl.BlockSpec((_BLK, _EMB), lambda i: (i, 0)),
            pl.BlockSpec((_BLK, _EMB), lambda i: (i, 0)),
            pl.BlockSpec((_BLK, _NUMF), lambda i: (i, 0)),
            pl.BlockSpec((_EMB, _HID), full),
            pl.BlockSpec((_EMB, _HID), full),
            pl.BlockSpec((_NUMF, _HID), full),
            pl.BlockSpec((1, _HID), full),
            pl.BlockSpec((_HID, _NCLS), full),
            pl.BlockSpec((1, _NCLS), full),
        ],
        out_specs=pl.BlockSpec((_BLK, _NCLS), lambda i: (i, 0)),
        out_shape=jax.ShapeDtypeStruct((_B, _NCLS), jnp.float32),
    )(s, t, n, w1s, w1t, w1n, b1, w2, b2)


def kernel(cat_feats, num_feats, src_emb, tgt_emb, W1, b1, W2, b2):
    src_id = cat_feats[:, 0]
    tgt_id = cat_feats[:, 1]
    srows, trows = _gather(src_id, tgt_id, src_emb, tgt_emb)
    s = srows.reshape(_B, _EMB)
    t = trows.reshape(_B, _EMB)
    w1s = W1[:, :_EMB].T
    w1t = W1[:, _EMB:2 * _EMB].T
    w1n = W1[:, 2 * _EMB:].T
    return _mlp(s, t, num_feats, w1s, w1t, w1n,
                b1.reshape(1, _HID), W2.T, b2.reshape(1, _NCLS))
